# Initial kernel scaffold; baseline (speedup 1.0000x reference)
import jax
import jax.numpy as jnp
from jax.experimental import pallas as pl


def kernel(x, edge_index, Wl, Wr, b):
    raise NotImplementedError("write your pallas kernel here")


# speedup vs baseline: 5.9770x; 5.9770x over previous
import functools

import jax
import jax.numpy as jnp
from jax import lax
from jax.experimental import pallas as pl
from jax.experimental.pallas import tpu as pltpu
from jax.experimental.pallas import tpu_sc as plsc

_NC = 2
_NS = 16
_CH = 128


@functools.lru_cache(maxsize=None)
def _build_segsum(N, H, E):
    NW = _NC * _NS
    nchunks = E // _CH
    assert E % _CH == 0
    per = nchunks // NW
    rem = nchunks % NW
    rpt = 8 * (N // (8 * _NS))
    tail = N - _NS * rpt
    mesh = plsc.VectorSubcoreMesh(core_axis_name="c", subcore_axis_name="s")

    @functools.partial(
        pl.kernel,
        mesh=mesh,
        out_type=jax.ShapeDtypeStruct((_NC, N, H), jnp.float32),
        scratch_types=[
            pltpu.VMEM((_CH,), jnp.int32),
            pltpu.VMEM((_CH,), jnp.int32),
            pltpu.VMEM((_CH, H), jnp.float32),
            pltpu.VMEM_SHARED((N, H), jnp.float32),
            pltpu.SemaphoreType.DMA,
        ],
    )
    def segsum(h_hbm, src_hbm, dst_hbm, zeros_hbm, out_hbm,
               sidx, didx, rows, acc, sem):
        cid = lax.axis_index("c")
        sid = lax.axis_index("s")
        wid = sid * _NC + cid
        r0 = sid * rpt
        pltpu.sync_copy(zeros_hbm.at[pl.ds(r0, rpt)], acc.at[pl.ds(r0, rpt)])
        if tail:
            @pl.when(sid == 0)
            def _():
                pltpu.sync_copy(zeros_hbm.at[pl.ds(_NS * rpt, tail)],
                                acc.at[pl.ds(_NS * rpt, tail)])
        plsc.subcore_barrier()

        def step(off):
            pltpu.sync_copy(src_hbm.at[pl.ds(off, _CH)], sidx)
            pltpu.async_copy(h_hbm.at[sidx], rows, sem).wait()
            pltpu.sync_copy(dst_hbm.at[pl.ds(off, _CH)], didx)
            pltpu.sync_copy(rows, acc.at[didx], add=True)

        def body(j, carry):
            step((wid + j * NW) * _CH)
            return carry

        lax.fori_loop(0, per, body, 0)

        @pl.when(wid < rem)
        def _():
            step((NW * per + wid) * _CH)

        plsc.subcore_barrier()
        pltpu.sync_copy(acc.at[pl.ds(r0, rpt)],
                        out_hbm.at[cid, pl.ds(r0, rpt)])
        if tail:
            @pl.when(sid == 0)
            def _():
                pltpu.sync_copy(acc.at[pl.ds(_NS * rpt, tail)],
                                out_hbm.at[cid, pl.ds(_NS * rpt, tail)])

    return segsum


@functools.lru_cache(maxsize=None)
def _build_count(N, E, W=128):
    NW = _NC * _NS
    nchunks = E // _CH
    per = nchunks // NW
    rem = nchunks % NW
    rpt = 8 * (N // (8 * _NS))
    tail = N - _NS * rpt
    mesh = plsc.VectorSubcoreMesh(core_axis_name="c", subcore_axis_name="s")

    @functools.partial(
        pl.kernel,
        mesh=mesh,
        out_type=jax.ShapeDtypeStruct((_NC, N, W), jnp.float32),
        scratch_types=[
            pltpu.VMEM((_CH,), jnp.int32),
            pltpu.VMEM((_CH, W), jnp.float32),
            pltpu.VMEM_SHARED((N, W), jnp.float32),
        ],
    )
    def count(dst_hbm, ones_hbm, zerosw_hbm, out_hbm, didx, ones_v, cacc):
        cid = lax.axis_index("c")
        sid = lax.axis_index("s")
        wid = sid * _NC + cid
        r0 = sid * rpt
        pltpu.sync_copy(ones_hbm, ones_v)
        pltpu.sync_copy(zerosw_hbm.at[pl.ds(r0, rpt)], cacc.at[pl.ds(r0, rpt)])
        if tail:
            @pl.when(sid == 0)
            def _():
                pltpu.sync_copy(zerosw_hbm.at[pl.ds(_NS * rpt, tail)],
                                cacc.at[pl.ds(_NS * rpt, tail)])
        plsc.subcore_barrier()

        def step(off):
            pltpu.sync_copy(dst_hbm.at[pl.ds(off, _CH)], didx)
            pltpu.sync_copy(ones_v, cacc.at[didx], add=True)

        def body(j, carry):
            step((wid + j * NW) * _CH)
            return carry

        lax.fori_loop(0, per, body, 0)

        @pl.when(wid < rem)
        def _():
            step((NW * per + wid) * _CH)

        plsc.subcore_barrier()
        pltpu.sync_copy(cacc.at[pl.ds(r0, rpt)],
                        out_hbm.at[cid, pl.ds(r0, rpt)])
        if tail:
            @pl.when(sid == 0)
            def _():
                pltpu.sync_copy(cacc.at[pl.ds(_NS * rpt, tail)],
                                out_hbm.at[cid, pl.ds(_NS * rpt, tail)])

    return count


def _fuse(p, h, cnt, Wl_i, Wr_i, b_i, relu):
    N, H = h.shape
    BR = 2000
    nb = N // BR

    def body(p_ref, h_ref, cnt_ref, wl_ref, wr_ref, b_ref, o_ref):
        inv = 1.0 / jnp.maximum(cnt_ref[...], 1.0)
        agg = (p_ref[0] + p_ref[1]) * inv
        acc = jnp.dot(agg, wl_ref[...], preferred_element_type=jnp.float32)
        acc = acc + jnp.dot(h_ref[...], wr_ref[...],
                            preferred_element_type=jnp.float32)
        acc = acc + b_ref[...]
        if relu:
            acc = jnp.maximum(acc, 0.0)
        o_ref[...] = acc

    return pl.pallas_call(
        body,
        grid=(nb,),
        in_specs=[
            pl.BlockSpec((2, BR, H), lambda i: (0, i, 0)),
            pl.BlockSpec((BR, H), lambda i: (i, 0)),
            pl.BlockSpec((BR, 1), lambda i: (i, 0)),
            pl.BlockSpec((H, H), lambda i: (0, 0)),
            pl.BlockSpec((H, H), lambda i: (0, 0)),
            pl.BlockSpec((1, H), lambda i: (0, 0)),
        ],
        out_specs=pl.BlockSpec((BR, H), lambda i: (i, 0)),
        out_shape=jax.ShapeDtypeStruct((N, H), jnp.float32),
    )(p, h, cnt, Wl_i, Wr_i, b_i.reshape(1, H))


def kernel(x, edge_index, Wl, Wr, b):
    N, D = x.shape
    E = edge_index.shape[1]
    L = Wl.shape[0]
    src = edge_index[0]
    dst = edge_index[1]
    zeros = jnp.zeros((N, D), jnp.float32)
    onesw = jnp.ones((_CH, D), jnp.float32)

    cparts = _build_count(N, E, D)(dst, onesw, zeros)
    cnt = (cparts[0, :, :1] + cparts[1, :, :1])

    segsum = _build_segsum(N, D, E)
    h = x
    for i in range(L):
        p = segsum(h, src, dst, zeros)
        h = _fuse(p, h, cnt, Wl[i], Wr[i], b[i], relu=(i < L - 1))
    return h
